# baseline (speedup 1.0000x reference)
import functools

import jax
import jax.numpy as jnp
from jax import lax
from jax.experimental import pallas as pl
from jax.experimental.pallas import tpu as pltpu
from jax.experimental.pallas import tpu_sc as plsc

B = 1024
SIZE = 26
D = 4096
NLEV = 100
LPAD = 104
TROWS = SIZE * LPAD

NC = 2
NS = 16
NW = NC * NS
B_PER_W = B // NW

NQ = 4
SLW = 8
LNW = 128
LANES = 16


def _table_body(pos_ref, lev_ref, t_ref):
    p4 = pos_ref[0]
    l4 = lev_ref[...]
    w = jnp.zeros((LPAD, SLW, LNW), jnp.int32)
    for q in range(NQ):
        p = p4[q]
        l = l4[:, q]
        t = p + l * (1.0 - 2.0 * p)
        w = w + t.astype(jnp.int32) * (1 << (8 * q))
    t_ref[...] = w


def _build_table(pos4, lev4):
    return pl.pallas_call(
        _table_body,
        grid=(SIZE,),
        in_specs=[
            pl.BlockSpec((1, NQ, SLW, LNW), lambda s: (s, 0, 0, 0)),
            pl.BlockSpec((LPAD, NQ, SLW, LNW), lambda s: (0, 0, 0, 0)),
        ],
        out_specs=pl.BlockSpec((LPAD, SLW, LNW), lambda s: (s, 0, 0)),
        out_shape=jax.ShapeDtypeStruct((TROWS, SLW, LNW), jnp.int32),
    )(pos4, lev4)


def _fidx_body(x_ref, out_ref):
    xv = x_ref[...]
    q = jnp.floor(xv * float(NLEV))
    q = jnp.clip(q, 0.0, float(NLEV - 1)).astype(jnp.int32)
    c = lax.broadcasted_iota(jnp.int32, (B // 2, 2 * SIZE), 1)
    out_ref[...] = q + (c % SIZE) * LPAD


def _build_fidx(x):
    x2 = x.reshape(B // 2, 2 * SIZE)
    return pl.pallas_call(
        _fidx_body,
        in_specs=[pl.BlockSpec((B // 2, 2 * SIZE), lambda: (0, 0))],
        out_specs=pl.BlockSpec((B // 2, 2 * SIZE), lambda: (0, 0)),
        out_shape=jax.ShapeDtypeStruct((B // 2, 2 * SIZE), jnp.int32),
    )(x2)


def _unpack_body(w_ref, out_ref):
    w = w_ref[...]
    blk = w.shape[0]
    for q in range(NQ):
        f = ((w >> (8 * q)) & 0xFF).astype(jnp.float32)
        out_ref[:, q * (D // NQ):(q + 1) * (D // NQ)] = f.reshape(
            blk, D // NQ)


def _unpack(acc_w):
    blk = 256
    return pl.pallas_call(
        _unpack_body,
        grid=(B // blk,),
        in_specs=[pl.BlockSpec((blk, SLW, LNW), lambda i: (i, 0, 0))],
        out_specs=pl.BlockSpec((blk, D), lambda i: (i, 0)),
        out_shape=jax.ShapeDtypeStruct((B, D), jnp.float32),
    )(acc_w)


def _sum_tree(vals, nchain=4):
    chains = [vals[i::nchain] for i in range(nchain)]
    totals = []
    for ch in chains:
        t = ch[0]
        for v in ch[1:]:
            t = t + v
        totals.append(t)
    while len(totals) > 1:
        nxt = [a + b for a, b in zip(totals[0::2], totals[1::2])]
        if len(totals) % 2:
            nxt.append(totals[-1])
        totals = nxt
    return totals[0]


def _sc_gather_sum(table_w, fidx):
    mesh = plsc.VectorSubcoreMesh(core_axis_name="c", subcore_axis_name="s")

    NP = B_PER_W // 2
    GP = 2
    NG = NP // GP

    @functools.partial(
        pl.kernel,
        mesh=mesh,
        out_type=jax.ShapeDtypeStruct((B, SLW, LNW), jnp.int32),
        scratch_types=[
            pltpu.VMEM((NP, 2 * SIZE), jnp.int32),
            pltpu.VMEM((2 * SIZE, SLW, LNW), jnp.int32),
            pltpu.VMEM((2 * SIZE, SLW, LNW), jnp.int32),
            pltpu.VMEM((2 * GP, SLW, LNW), jnp.int32),
            pltpu.SemaphoreType.DMA,
            pltpu.SemaphoreType.DMA,
        ],
    )
    def k(table_hbm, fidx_hbm, out_hbm, idx_v, rows_a, rows_b, outblk_v,
          sem_a, sem_b):
        wid = lax.axis_index("s") * NC + lax.axis_index("c")
        base = wid * B_PER_W
        pltpu.sync_copy(fidx_hbm.at[pl.ds(wid * NP, NP)], idx_v)

        bufs = (rows_a, rows_b)
        sems = (sem_a, sem_b)

        def accumulate(rows_v, h, jj):
            def chunk_body(c, carry2):
                off = c * LANES
                for sl in range(SLW):
                    acc = _sum_tree(
                        [rows_v[SIZE * h + s, sl, pl.ds(off, LANES)]
                         for s in range(SIZE)])
                    outblk_v[jj, sl, pl.ds(off, LANES)] = acc
                return carry2

            lax.fori_loop(0, LNW // LANES, chunk_body, 0, unroll=False)

        pltpu.async_copy(table_hbm.at[idx_v.at[0]], rows_a, sem_a)

        def group(g, carry):
            for pp in range(GP):
                p = GP * g + pp
                cur, nxt = bufs[pp % 2], bufs[1 - pp % 2]
                csem, nsem = sems[pp % 2], sems[1 - pp % 2]
                pltpu.async_copy(
                    table_hbm.at[idx_v.at[jnp.minimum(p + 1, NP - 1)]],
                    nxt, nsem)
                pltpu.make_async_copy(table_hbm.at[idx_v.at[p]], cur,
                                      csem).wait()
                accumulate(cur, 0, 2 * pp)
                accumulate(cur, 1, 2 * pp + 1)
            pltpu.sync_copy(outblk_v, out_hbm.at[pl.ds(base + g * 2 * GP,
                                                       2 * GP)])
            return carry

        lax.fori_loop(0, NG, group, 0, unroll=False)
        pltpu.make_async_copy(table_hbm.at[idx_v.at[NP - 1]], rows_a,
                              sem_a).wait()

    return k(table_w, fidx)


def kernel(x, position, levels):
    levels_pad = jnp.pad(levels, ((0, LPAD - NLEV), (0, 0)))
    pos4 = position.reshape(SIZE, NQ, SLW, LNW)
    lev4 = levels_pad.reshape(LPAD, NQ, SLW, LNW)
    table_w = _build_table(pos4, lev4)
    fidx = _build_fidx(x)
    acc_w = _sc_gather_sum(table_w, fidx)
    return _unpack(acc_w)


# speedup vs baseline: 1.0412x; 1.0412x over previous
import functools

import jax
import jax.numpy as jnp
from jax import lax
from jax.experimental import pallas as pl
from jax.experimental.pallas import tpu as pltpu
from jax.experimental.pallas import tpu_sc as plsc

B = 1024
SIZE = 26
D = 4096
NLEV = 100
LPAD = 104
TROWS = SIZE * LPAD

NC = 2
NS = 16
NW = NC * NS
B_PER_W = B // NW

NQ = 4
SLW = 8
LNW = 128
LANES = 16


def _table_body(pos_ref, lev_ref, t_ref):
    p4 = pos_ref[0]
    l4 = lev_ref[...]
    w = jnp.zeros((LPAD, SLW, LNW), jnp.int32)
    for q in range(NQ):
        p = p4[q]
        l = l4[:, q]
        t = p + l * (1.0 - 2.0 * p)
        w = w + t.astype(jnp.int32) * (1 << (8 * q))
    t_ref[...] = w


def _build_table(pos4, lev4):
    return pl.pallas_call(
        _table_body,
        grid=(SIZE,),
        in_specs=[
            pl.BlockSpec((1, NQ, SLW, LNW), lambda s: (s, 0, 0, 0)),
            pl.BlockSpec((LPAD, NQ, SLW, LNW), lambda s: (0, 0, 0, 0)),
        ],
        out_specs=pl.BlockSpec((LPAD, SLW, LNW), lambda s: (s, 0, 0)),
        out_shape=jax.ShapeDtypeStruct((TROWS, SLW, LNW), jnp.int32),
    )(pos4, lev4)


def _fidx_body(x_ref, out_ref):
    xv = x_ref[...]
    q = jnp.floor(xv * float(NLEV))
    q = jnp.clip(q, 0.0, float(NLEV - 1)).astype(jnp.int32)
    s = lax.broadcasted_iota(jnp.int32, (B, SIZE), 1)
    out_ref[...] = q + s * LPAD


def _build_fidx(x):
    return pl.pallas_call(
        _fidx_body,
        in_specs=[pl.BlockSpec((B, SIZE), lambda: (0, 0))],
        out_specs=pl.BlockSpec((B, SIZE), lambda: (0, 0)),
        out_shape=jax.ShapeDtypeStruct((B, SIZE), jnp.int32),
    )(x)


def _unpack_body(w_ref, out_ref):
    w = w_ref[...]
    blk = w.shape[0]
    for q in range(NQ):
        f = ((w >> (8 * q)) & 0xFF).astype(jnp.float32)
        out_ref[:, q * (D // NQ):(q + 1) * (D // NQ)] = f.reshape(
            blk, D // NQ)


def _unpack(acc_w):
    blk = 256
    return pl.pallas_call(
        _unpack_body,
        grid=(B // blk,),
        in_specs=[pl.BlockSpec((blk, SLW, LNW), lambda i: (i, 0, 0))],
        out_specs=pl.BlockSpec((blk, D), lambda i: (i, 0)),
        out_shape=jax.ShapeDtypeStruct((B, D), jnp.float32),
    )(acc_w)


def _sum_tree(vals, nchain=4):
    chains = [vals[i::nchain] for i in range(nchain)]
    totals = []
    for ch in chains:
        t = ch[0]
        for v in ch[1:]:
            t = t + v
        totals.append(t)
    while len(totals) > 1:
        nxt = [a + b for a, b in zip(totals[0::2], totals[1::2])]
        if len(totals) % 2:
            nxt.append(totals[-1])
        totals = nxt
    return totals[0]


def _sc_gather_sum(table_w, fidx):
    mesh = plsc.VectorSubcoreMesh(core_axis_name="c", subcore_axis_name="s")

    NBUF = 3

    @functools.partial(
        pl.kernel,
        mesh=mesh,
        out_type=jax.ShapeDtypeStruct((B, SLW, LNW), jnp.int32),
        scratch_types=[
            pltpu.VMEM((B_PER_W, SIZE), jnp.int32),
            pltpu.VMEM((SIZE, SLW, LNW), jnp.int32),
            pltpu.VMEM((SIZE, SLW, LNW), jnp.int32),
            pltpu.VMEM((SIZE, SLW, LNW), jnp.int32),
            pltpu.VMEM((SLW, LNW), jnp.int32),
            pltpu.SemaphoreType.DMA,
            pltpu.SemaphoreType.DMA,
            pltpu.SemaphoreType.DMA,
        ],
    )
    def k(table_hbm, fidx_hbm, out_hbm, idx_v, rows_a, rows_b, rows_c,
          outrow_v, sem_a, sem_b, sem_c):
        wid = lax.axis_index("s") * NC + lax.axis_index("c")
        base = wid * B_PER_W
        pltpu.sync_copy(fidx_hbm.at[pl.ds(base, B_PER_W)], idx_v)

        bufs = (rows_a, rows_b, rows_c)
        sems = (sem_a, sem_b, sem_c)

        def accumulate(rows_v, j):
            def chunk_body(c, carry2):
                off = c * LANES
                for sl in range(SLW):
                    acc = _sum_tree(
                        [rows_v[s, sl, pl.ds(off, LANES)]
                         for s in range(SIZE)])
                    outrow_v[sl, pl.ds(off, LANES)] = acc
                return carry2

            lax.fori_loop(0, LNW // LANES, chunk_body, 0, unroll=False)
            pltpu.sync_copy(outrow_v, out_hbm.at[base + j])

        for r in range(NBUF - 1):
            pltpu.async_copy(table_hbm.at[idx_v.at[r]], bufs[r], sems[r])

        def step(i, carry):
            j = NBUF * i
            for r in range(NBUF):
                jj = j + r
                pltpu.make_async_copy(table_hbm.at[idx_v.at[jj]], bufs[r],
                                      sems[r]).wait()
                nxt = jnp.minimum(jj + NBUF - 1, B_PER_W - 1)
                pltpu.async_copy(table_hbm.at[idx_v.at[nxt]],
                                 bufs[(r + NBUF - 1) % NBUF],
                                 sems[(r + NBUF - 1) % NBUF])
                accumulate(bufs[r], jj)
            return carry

        lax.fori_loop(0, B_PER_W // NBUF, step, 0, unroll=False)
        tail = B_PER_W - (B_PER_W // NBUF) * NBUF
        for r in range(tail):
            jj = B_PER_W - tail + r
            pltpu.make_async_copy(table_hbm.at[idx_v.at[jj]],
                                  bufs[r], sems[r]).wait()
            accumulate(bufs[r], jj)

    return k(table_w, fidx)


def kernel(x, position, levels):
    levels_pad = jnp.pad(levels, ((0, LPAD - NLEV), (0, 0)))
    pos4 = position.reshape(SIZE, NQ, SLW, LNW)
    lev4 = levels_pad.reshape(LPAD, NQ, SLW, LNW)
    table_w = _build_table(pos4, lev4)
    fidx = _build_fidx(x)
    acc_w = _sc_gather_sum(table_w, fidx)
    return _unpack(acc_w)
